# baseline (speedup 1.0000x reference)
import functools

import jax
from jax import lax
import jax.numpy as jnp
from jax.experimental import pallas as pl
from jax.experimental.pallas import tpu as pltpu
from jax.experimental.pallas import tpu_sc as plsc

_NUM_SUBCORES = 16


def _half_gather(x_half, table, tag):
    b, l = x_half.shape
    _, d = table.shape
    nw = _NUM_SUBCORES
    b_per_w = b // nw
    chunk = 8
    nchunks = b_per_w // chunk

    mesh = plsc.VectorSubcoreMesh(
        core_axis_name="c", subcore_axis_name="s", num_cores=1
    )

    @functools.partial(
        pl.kernel,
        mesh=mesh,
        out_type=jax.ShapeDtypeStruct((b, l, d), table.dtype),
        scratch_types=[
            pltpu.VMEM((b_per_w, l), jnp.int32),
            pltpu.VMEM((chunk, l, d), table.dtype),
            pltpu.VMEM((chunk, l, d), table.dtype),
            pltpu.SemaphoreType.DMA,
            pltpu.SemaphoreType.DMA,
            pltpu.SemaphoreType.DMA,
            pltpu.SemaphoreType.DMA,
        ],
    )
    def gather_kernel(tab_hbm, x_hbm, o_hbm, idx_v, buf0, buf1, g0, g1, o0, o1):
        wid = lax.axis_index("s")
        base = wid * b_per_w
        pltpu.sync_copy(x_hbm.at[pl.ds(base, b_per_w)], idx_v)

        bufs = (buf0, buf1)
        gsems = (g0, g1)
        osems = (o0, o1)

        @pl.loop(0, nchunks, step=2)
        def _(g):
            for bi in range(2):
                buf, gsem, osem = bufs[bi], gsems[bi], osems[bi]
                gg = g + bi

                @pl.when(gg >= 2)
                def _():
                    pltpu.make_async_copy(
                        buf, o_hbm.at[pl.ds(base, chunk)], osem
                    ).wait()

                copies = [
                    pltpu.async_copy(
                        tab_hbm.at[idx_v.at[gg * chunk + r]], buf.at[r], gsem
                    )
                    for r in range(chunk)
                ]
                for cp in copies:
                    cp.wait()
                pltpu.async_copy(
                    buf, o_hbm.at[pl.ds(base + gg * chunk, chunk)], osem
                )

        for bi in range(2):
            pltpu.make_async_copy(
                bufs[bi], o_hbm.at[pl.ds(base, chunk)], osems[bi]
            ).wait()

    return gather_kernel(table, x_half)


def kernel(x, table):
    b = x.shape[0]
    h = b // 2
    o0 = _half_gather(x[:h], table, 0)
    o1 = _half_gather(x[h:], table, 1)
    return jnp.concatenate([o0, o1], axis=0)


# speedup vs baseline: 1.5144x; 1.5144x over previous
import functools

import jax
from jax import lax
import jax.numpy as jnp
from jax.experimental import pallas as pl
from jax.experimental.pallas import tpu as pltpu
from jax.experimental.pallas import tpu_sc as plsc

_NUM_SUBCORES = 16


def kernel(x, table):
    b, l = x.shape
    _, d = table.shape
    nw = _NUM_SUBCORES
    b_per_w = b // nw
    chunk = 8
    nchunks = b_per_w // chunk
    assert b_per_w * nw == b and chunk * nchunks == b_per_w and nchunks % 2 == 0

    mesh = plsc.VectorSubcoreMesh(
        core_axis_name="c", subcore_axis_name="s", num_cores=1
    )

    @functools.partial(
        pl.kernel,
        mesh=mesh,
        out_type=jax.ShapeDtypeStruct((b, l, d), table.dtype),
        scratch_types=[
            pltpu.VMEM((chunk, l), jnp.int32),
            pltpu.VMEM((chunk, l), jnp.int32),
            pltpu.VMEM((chunk, l, d), table.dtype),
            pltpu.VMEM((chunk, l, d), table.dtype),
            pltpu.SemaphoreType.DMA,
            pltpu.SemaphoreType.DMA,
            pltpu.SemaphoreType.DMA,
            pltpu.SemaphoreType.DMA,
            pltpu.SemaphoreType.DMA,
            pltpu.SemaphoreType.DMA,
        ],
    )
    def gather_kernel(
        tab_hbm, x_hbm, o_hbm, ib0, ib1, buf0, buf1, i0, i1, g0, g1, o0, o1
    ):
        wid = lax.axis_index("s")
        base = wid * b_per_w

        ibufs = (ib0, ib1)
        bufs = (buf0, buf1)
        isems = (i0, i1)
        gsems = (g0, g1)
        osems = (o0, o1)

        for bi in range(2):
            pltpu.async_copy(
                x_hbm.at[pl.ds(base + bi * chunk, chunk)], ibufs[bi], isems[bi]
            )

        @pl.loop(0, nchunks, step=2)
        def _(g):
            for bi in range(2):
                ibuf, buf = ibufs[bi], bufs[bi]
                isem, gsem, osem = isems[bi], gsems[bi], osems[bi]
                gg = g + bi

                @pl.when(gg >= 2)
                def _():
                    pltpu.make_async_copy(
                        buf, o_hbm.at[pl.ds(base, chunk)], osem
                    ).wait()

                pltpu.make_async_copy(
                    x_hbm.at[pl.ds(base, chunk)], ibuf, isem
                ).wait()
                copies = [
                    pltpu.async_copy(tab_hbm.at[ibuf.at[r]], buf.at[r], gsem)
                    for r in range(chunk)
                ]
                for cp in copies:
                    cp.wait()

                @pl.when(gg + 2 < nchunks)
                def _():
                    pltpu.async_copy(
                        x_hbm.at[pl.ds(base + (gg + 2) * chunk, chunk)],
                        ibuf,
                        isem,
                    )

                pltpu.async_copy(
                    buf, o_hbm.at[pl.ds(base + gg * chunk, chunk)], osem
                )

        for bi in range(2):
            pltpu.make_async_copy(
                bufs[bi], o_hbm.at[pl.ds(base, chunk)], osems[bi]
            ).wait()

    return gather_kernel(table, x)
